# baseline (speedup 1.0000x reference)
import jax
import jax.numpy as jnp
from jax import lax
from jax.experimental import pallas as pl
from jax.experimental.pallas import tpu as pltpu
from jax.experimental.pallas import tpu_sc as plsc

_SIZE = 16384
_INPUT = 4096
_K = 25

_L = 16
_NCORE = 2
_NSUB = 16
_NW = _NCORE * _NSUB

_R_SC = 5120
_R_TC = _SIZE - _R_SC
_ROWS_W = _R_SC // _NW
_CH = 4
_NCHUNK = _ROWS_W // _CH

_BR = 1024
_NB_T = _R_TC // _BR
_PAD_N = 48


def _tc_body(t_ref, perms_ref, out_ref):
    blk = perms_ref[...]
    mask = (blk > t_ref[...]).astype(jnp.float32)
    out_ref[0, 0, :] = jnp.sum(mask, axis=1)


def _sc_body(t_hbm, perms_hbm, out_hbm, t_v, buf, cnt_v, pad, sem0, sem1):
    cidx = lax.axis_index("c")
    sidx = lax.axis_index("s")
    wid = cidx * _NSUB + sidx
    row0 = _R_TC + wid * _ROWS_W
    sems = (sem0, sem1)
    iota16 = lax.iota(jnp.int32, _L)
    zeros16 = jnp.zeros((_L,), jnp.float32)
    pltpu.sync_copy(t_hbm, t_v)
    for z in range(_PAD_N // _L):
        pad[pl.ds(z * _L, _L)] = zeros16

    def lane_sum(vec):
        s = vec
        for sh in (8, 4, 2, 1):
            pad[pl.ds(0, _L)] = s
            s = s + pad[pl.ds(sh, _L)]
        return s

    for b in range(2):
        pltpu.async_copy(
            perms_hbm.at[pl.ds(row0 + b * _CH, _CH)], buf.at[b], sems[b])

    def outer(gg, carry):
        contrib = jnp.zeros((_L,), jnp.float32)
        for bb in range(_L // _CH):
            b = bb % 2
            g = gg * (_L // _CH) + bb
            pltpu.make_async_copy(
                perms_hbm.at[pl.ds(row0 + g * _CH, _CH)], buf.at[b],
                sems[b]).wait()

            def col(ci, accs):
                tv = t_v[pl.ds(ci * _L, _L)]
                return tuple(
                    accs[r]
                    + jnp.where(buf[b, r, pl.ds(ci * _L, _L)] > tv,
                                jnp.float32(1), jnp.float32(0))
                    for r in range(_CH))

            accs = lax.fori_loop(
                0, _INPUT // _L, col,
                tuple(jnp.zeros((_L,), jnp.float32) for _ in range(_CH)))
            for r in range(_CH):
                k = bb * _CH + r
                t0 = jnp.where(iota16 == 0, lane_sum(accs[r]),
                               jnp.float32(0))
                if k == 0:
                    contrib = contrib + t0
                else:
                    pad[pl.ds(32, _L)] = t0
                    contrib = contrib + pad[pl.ds(32 - k, _L)]

            @pl.when(g + 2 < _NCHUNK)
            def _prefetch():
                pltpu.async_copy(
                    perms_hbm.at[pl.ds(row0 + (g + 2) * _CH, _CH)],
                    buf.at[b], sems[b])
        cnt_v[pl.ds(gg * _L, _L)] = contrib
        return carry

    lax.fori_loop(0, _NCHUNK // (_L // _CH), outer, 0)
    pltpu.sync_copy(cnt_v, out_hbm.at[pl.ds(wid * _ROWS_W, _ROWS_W)])


def _merge_body(ctc_ref, csc_ref, mtc_ref, msc_ref):
    ctc = ctc_ref[...].reshape(_NB_T, _BR)
    csc = csc_ref[...]

    def step(_, carry):
        lo, hi = carry
        mid = (lo + hi) // 2
        midf = mid.astype(jnp.float32)
        cnt = (jnp.sum((ctc >= midf).astype(jnp.int32))
               + jnp.sum((csc >= midf).astype(jnp.int32)))
        ok = cnt >= _K + 1
        return jnp.where(ok, mid, lo), jnp.where(ok, hi, mid)

    lo, _ = lax.fori_loop(0, 13, step, (jnp.int32(0), jnp.int32(_INPUT + 1)))
    thr = lo.astype(jnp.float32)
    mtc_ref[...] = (ctc > thr).astype(jnp.int32).reshape(_NB_T, 1, _BR)
    msc_ref[...] = (csc > thr).astype(jnp.int32)


def kernel(input_vector, perms):
    t1d = jnp.where(input_vector, jnp.float32(0.6), jnp.inf)
    t2d = t1d.reshape(1, _INPUT)

    mesh = plsc.VectorSubcoreMesh(core_axis_name="c", subcore_axis_name="s")
    counts_sc = pl.kernel(
        _sc_body,
        out_type=jax.ShapeDtypeStruct((_R_SC,), jnp.float32),
        mesh=mesh,
        scratch_types=[
            pltpu.VMEM((_INPUT,), jnp.float32),
            pltpu.VMEM((2, _CH, _INPUT), jnp.float32),
            pltpu.VMEM((_ROWS_W,), jnp.float32),
            pltpu.VMEM((_PAD_N,), jnp.float32),
            pltpu.SemaphoreType.DMA,
            pltpu.SemaphoreType.DMA,
        ],
    )(t1d, perms)

    counts_tc = pl.pallas_call(
        _tc_body,
        grid=(_NB_T,),
        in_specs=[
            pl.BlockSpec((1, _INPUT), lambda i: (0, 0)),
            pl.BlockSpec((_BR, _INPUT), lambda i: (i, 0)),
        ],
        out_specs=pl.BlockSpec((1, 1, _BR), lambda i: (i, 0, 0)),
        out_shape=jax.ShapeDtypeStruct((_NB_T, 1, _BR), jnp.float32),
    )(t2d, perms)

    mtc, msc = pl.pallas_call(
        _merge_body,
        grid=(1,),
        in_specs=[
            pl.BlockSpec((_NB_T, 1, _BR), lambda i: (0, 0, 0)),
            pl.BlockSpec((_R_SC,), lambda i: (0,)),
        ],
        out_specs=[
            pl.BlockSpec((_NB_T, 1, _BR), lambda i: (0, 0, 0)),
            pl.BlockSpec((_R_SC,), lambda i: (0,)),
        ],
        out_shape=[
            jax.ShapeDtypeStruct((_NB_T, 1, _BR), jnp.int32),
            jax.ShapeDtypeStruct((_R_SC,), jnp.int32),
        ],
    )(counts_tc, counts_sc)

    out = jnp.concatenate([mtc.reshape(-1), msc])
    return out.astype(jnp.bool_)


# speedup vs baseline: 1.2316x; 1.2316x over previous
import jax
import jax.numpy as jnp
from jax.experimental import pallas as pl
from jax.experimental.pallas import tpu as pltpu

_SIZE = 16384
_INPUT = 4096
_K = 25
_BR = 1024
_NB = _SIZE // _BR


def _body(t_ref, perms_ref, out_ref, ov_ref):
    i = pl.program_id(0)
    blk = perms_ref[...]
    mask = (blk > t_ref[...]).astype(jnp.float32)
    ov_ref[i, :] = jnp.sum(mask, axis=1)

    @pl.when(i == _NB - 1)
    def _finish():
        ovs = ov_ref[...]

        def step(_, carry):
            lo, hi = carry
            mid = (lo + hi) // 2
            cnt = jnp.sum((ovs >= mid.astype(jnp.float32)).astype(jnp.int32))
            ok = cnt >= _K + 1
            return jnp.where(ok, mid, lo), jnp.where(ok, hi, mid)

        lo, _ = jax.lax.fori_loop(
            0, 13, step, (jnp.int32(0), jnp.int32(_INPUT + 1)))
        out_ref[...] = (ovs > lo.astype(jnp.float32)).astype(jnp.int32)


def kernel(input_vector, perms):
    thresholds = jnp.where(input_vector, jnp.float32(0.6), jnp.inf)
    thresholds = thresholds.reshape(1, _INPUT)
    out = pl.pallas_call(
        _body,
        grid=(_NB,),
        in_specs=[
            pl.BlockSpec((1, _INPUT), lambda i: (0, 0)),
            pl.BlockSpec((_BR, _INPUT), lambda i: (i, 0)),
        ],
        out_specs=pl.BlockSpec((_NB, _BR), lambda i: (0, 0)),
        out_shape=jax.ShapeDtypeStruct((_NB, _BR), jnp.int32),
        scratch_shapes=[pltpu.VMEM((_NB, _BR), jnp.float32)],
    )(thresholds, perms)
    return out.reshape(_SIZE).astype(jnp.bool_)
